# baseline (speedup 1.0000x reference)
import jax
import jax.numpy as jnp
from jax.experimental import pallas as pl

V = 128
HID = 64
M = 8
N_P = 4
B = 16
NR = (M + N_P) * B


def _leaky(x):
    return jnp.where(x >= 0, x, 0.2 * x)


def _iota2(shape, dim):
    return jax.lax.broadcasted_iota(jnp.int32, shape, dim)


def _body(sig_ref, w1r_ref, fc1b_ref, fc2W_ref, fc2b_ref,
          attnW_ref, attna_ref, Wih_ref, Whh_ref, bih_ref, bhh_ref,
          outW_ref, outb_ref, out_ref):
    f32 = jnp.float32
    bf16 = jnp.bfloat16
    hi = jax.lax.Precision.HIGHEST

    sig = sig_ref[...]
    w1r = w1r_ref[...]
    fc1b = fc1b_ref[...]
    fc2W = fc2W_ref[...]
    fc2b = fc2b_ref[...]
    attnW = attnW_ref[...]
    attna = attna_ref[...]
    Wih = Wih_ref[...]
    Whh = Whh_ref[...]
    bih = bih_ref[...]
    bhh = bhh_ref[...]
    outW = outW_ref[...]
    outb = outb_ref[...]

    def dotb(x, W):
        return jax.lax.dot_general(
            x.astype(bf16), W.astype(bf16),
            (((x.ndim - 1,), (1,)), ((), ())),
            preferred_element_type=f32)

    P1 = jnp.where(_iota2((V, 8), 0) // 16 == _iota2((V, 8), 1),
                   f32(1.0 / 16.0), f32(0.0))
    P2 = jnp.where(_iota2((V, 8), 0) % 8 == _iota2((V, 8), 1),
                   f32(1.0 / 16.0), f32(0.0))
    m1 = jnp.dot(sig, P1, precision=hi, preferred_element_type=f32)
    m2 = jnp.dot(sig, P2, precision=hi, preferred_element_type=f32)

    h1_0 = m1[:, :, None] * w1r[None] + fc1b[None]
    h1_1 = m2[:, :, None] * w1r[None] + fc1b[None]
    G0 = dotb(h1_0, fc2W) + fc2b[None]
    G1 = dotb(h1_1, fc2W) + fc2b[None]

    X0 = jnp.concatenate(
        [jnp.broadcast_to(G0[:, q:q + 1, :], (NR, 8, HID)) for q in range(8)],
        axis=1)
    X1 = jnp.concatenate([G1] * 8, axis=1)

    refm = 0.5 * (X0 + X1)
    refw = dotb(refm, attnW)
    X0W = dotb(X0, attnW)
    X1W = dotb(X1, attnW)
    a1 = attna[:, :HID]
    a2 = attna[:, HID:]
    zx0 = dotb(X0W, a1)
    zx1 = dotb(X1W, a1)
    zr = dotb(refw, a2)
    z0 = _leaky(zx0 + zr)
    z1 = _leaky(zx1 + zr)
    al0 = jax.nn.sigmoid(z0 - z1)
    Y = al0 * X0 + (1.0 - al0) * X1

    gb = (bih + bhh)[None]

    def run_lstm(off, T):
        h = jnp.zeros((B, 64, HID), f32)
        cc = jnp.zeros((B, 64, HID), f32)
        for t in range(T):
            x_t = Y[off + t * B: off + (t + 1) * B]
            gates = dotb(x_t, Wih) + gb + dotb(h, Whh)
            i = jax.nn.sigmoid(gates[..., 0 * HID:1 * HID])
            f = jax.nn.sigmoid(gates[..., 1 * HID:2 * HID])
            g = jnp.tanh(gates[..., 2 * HID:3 * HID])
            o = jax.nn.sigmoid(gates[..., 3 * HID:4 * HID])
            cc = f * cc + i * g
            h = o * jnp.tanh(cc)
        return h

    h_t = run_lstm(0, M)
    h_p = run_lstm(M * B, N_P)

    o1 = dotb(h_t, outW[:, :HID])
    o2 = dotb(h_p, outW[:, HID:])
    O = jnp.sum(o1 + o2 + outb[0, 0], axis=2)

    vcol = _iota2((64, V), 1)
    Xp = jnp.where(8 * (vcol // 16) + vcol % 8 == _iota2((64, V), 0),
                   f32(1.0), f32(0.0))
    out_ref[...] = jnp.dot(O, Xp, precision=hi, preferred_element_type=f32)


def kernel(tendency, periodicity, fc1_W, fc1_b, fc2_W, fc2_b, attn_W, attn_a,
           lstm_Wih, lstm_Whh, lstm_bih, lstm_bhh, out_W, out_b):
    f32 = jnp.float32
    sig = jnp.concatenate([
        jnp.transpose(tendency, (1, 0, 2)).reshape(M * B, V),
        jnp.transpose(periodicity, (1, 0, 2)).reshape(N_P * B, V)],
        axis=0).astype(f32)
    args = (
        sig,
        fc1_W.reshape(1, HID).astype(f32),
        fc1_b.reshape(1, HID).astype(f32),
        fc2_W.astype(f32),
        fc2_b.reshape(1, HID).astype(f32),
        attn_W.astype(f32),
        attn_a.reshape(1, 2 * HID).astype(f32),
        lstm_Wih.astype(f32),
        lstm_Whh.astype(f32),
        lstm_bih.reshape(1, 4 * HID).astype(f32),
        lstm_bhh.reshape(1, 4 * HID).astype(f32),
        out_W.reshape(1, 2 * HID).astype(f32),
        out_b.reshape(1, 1).astype(f32),
    )
    out = pl.pallas_call(
        _body,
        out_shape=jax.ShapeDtypeStruct((B, V), f32),
    )(*args)
    return out


# speedup vs baseline: 1.1177x; 1.1177x over previous
import jax
import jax.numpy as jnp
from jax.experimental import pallas as pl

V = 128
HID = 64
M = 8
N_P = 4
B = 16
NR = (M + N_P) * B


def _leaky(x):
    return jnp.where(x >= 0, x, 0.2 * x)


def _iota2(shape, dim):
    return jax.lax.broadcasted_iota(jnp.int32, shape, dim)


def _body(sig_ref, w1r_ref, fc1b_ref, fc2W_ref, fc2b_ref,
          attnW_ref, attna_ref, Wih_ref, Whh_ref, bih_ref, bhh_ref,
          outW_ref, outb_ref, out_ref):
    f32 = jnp.float32
    bf16 = jnp.bfloat16
    hi = jax.lax.Precision.HIGHEST

    sig = sig_ref[...]
    w1r = w1r_ref[...]
    fc1b = fc1b_ref[...]
    fc2W = fc2W_ref[...]
    fc2b = fc2b_ref[...]
    attnW = attnW_ref[...]
    attna = attna_ref[...]
    Wih = Wih_ref[...]
    Whh = Whh_ref[...]
    bih = bih_ref[...]
    bhh = bhh_ref[...]
    outW = outW_ref[...]
    outb = outb_ref[...]

    def dotb(x, W):
        return jax.lax.dot_general(
            x.astype(bf16), W.astype(bf16),
            (((x.ndim - 1,), (1,)), ((), ())),
            preferred_element_type=f32)

    P1 = jnp.where(_iota2((V, 8), 0) // 16 == _iota2((V, 8), 1),
                   f32(1.0 / 16.0), f32(0.0))
    P2 = jnp.where(_iota2((V, 8), 0) % 8 == _iota2((V, 8), 1),
                   f32(1.0 / 16.0), f32(0.0))
    m1 = jnp.dot(sig, P1, precision=hi, preferred_element_type=f32)
    m2 = jnp.dot(sig, P2, precision=hi, preferred_element_type=f32)

    h1_0 = m1[:, :, None] * w1r[None] + fc1b[None]
    h1_1 = m2[:, :, None] * w1r[None] + fc1b[None]
    G0 = dotb(h1_0, fc2W) + fc2b[None]
    G1 = dotb(h1_1, fc2W) + fc2b[None]

    X0 = jnp.concatenate(
        [jnp.broadcast_to(G0[:, q:q + 1, :], (NR, 8, HID)) for q in range(8)],
        axis=1)
    X1 = jnp.concatenate([G1] * 8, axis=1)

    refm = 0.5 * (X0 + X1)
    refw = dotb(refm, attnW)
    a1 = attna[:, :HID]
    a2 = attna[:, HID:]
    zx0s = dotb(dotb(G0, attnW), a1)
    zx1s = dotb(dotb(G1, attnW), a1)
    zx0 = jnp.concatenate(
        [jnp.broadcast_to(zx0s[:, q:q + 1, :], (NR, 8, 1)) for q in range(8)],
        axis=1)
    zx1 = jnp.concatenate([zx1s] * 8, axis=1)
    zr = dotb(refw, a2)
    z0 = _leaky(zx0 + zr)
    z1 = _leaky(zx1 + zr)
    al0 = jax.nn.sigmoid(z0 - z1)
    Y = al0 * X0 + (1.0 - al0) * X1

    gb = (bih + bhh)[None]

    xW = dotb(Y, Wih) + gb

    h = jnp.zeros((2 * B, 64, HID), f32)
    cc = jnp.zeros((2 * B, 64, HID), f32)

    def lstm_update(gates, cc):
        i = jax.nn.sigmoid(gates[..., 0 * HID:1 * HID])
        f = jax.nn.sigmoid(gates[..., 1 * HID:2 * HID])
        g = jnp.tanh(gates[..., 2 * HID:3 * HID])
        o = jax.nn.sigmoid(gates[..., 3 * HID:4 * HID])
        cc = f * cc + i * g
        return o * jnp.tanh(cc), cc

    for t in range(4):
        x_t = xW[t * B:(t + 1) * B]
        gates = x_t + dotb(h[:B], Whh)
        h_new, cc_new = lstm_update(gates, cc[:B])
        h = jnp.concatenate([h_new, h[B:]], axis=0)
        cc = jnp.concatenate([cc_new, cc[B:]], axis=0)
    for t in range(4, 8):
        x_t = jnp.concatenate(
            [xW[t * B:(t + 1) * B],
             xW[M * B + (t - 4) * B: M * B + (t - 3) * B]], axis=0)
        gates = x_t + dotb(h, Whh)
        h, cc = lstm_update(gates, cc)

    h_t = h[:B]
    h_p = h[B:]

    o1 = dotb(h_t, outW[:, :HID])
    o2 = dotb(h_p, outW[:, HID:])
    O = jnp.sum(o1 + o2 + outb[0, 0], axis=2)

    vcol = _iota2((64, V), 1)
    Xp = jnp.where(8 * (vcol // 16) + vcol % 8 == _iota2((64, V), 0),
                   f32(1.0), f32(0.0))
    out_ref[...] = jnp.dot(O, Xp, precision=hi, preferred_element_type=f32)


def kernel(tendency, periodicity, fc1_W, fc1_b, fc2_W, fc2_b, attn_W, attn_a,
           lstm_Wih, lstm_Whh, lstm_bih, lstm_bhh, out_W, out_b):
    f32 = jnp.float32
    sig = jnp.concatenate([
        jnp.transpose(tendency, (1, 0, 2)).reshape(M * B, V),
        jnp.transpose(periodicity, (1, 0, 2)).reshape(N_P * B, V)],
        axis=0).astype(f32)
    args = (
        sig,
        fc1_W.reshape(1, HID).astype(f32),
        fc1_b.reshape(1, HID).astype(f32),
        fc2_W.astype(f32),
        fc2_b.reshape(1, HID).astype(f32),
        attn_W.astype(f32),
        attn_a.reshape(1, 2 * HID).astype(f32),
        lstm_Wih.astype(f32),
        lstm_Whh.astype(f32),
        lstm_bih.reshape(1, 4 * HID).astype(f32),
        lstm_bhh.reshape(1, 4 * HID).astype(f32),
        out_W.reshape(1, 2 * HID).astype(f32),
        out_b.reshape(1, 1).astype(f32),
    )
    out = pl.pallas_call(
        _body,
        out_shape=jax.ShapeDtypeStruct((B, V), f32),
    )(*args)
    return out


# speedup vs baseline: 1.1227x; 1.0045x over previous
import jax
import jax.numpy as jnp
from jax.experimental import pallas as pl

V = 128
HID = 64
M = 8
N_P = 4
B = 16
NR = (M + N_P) * B


def _leaky(x):
    return jnp.where(x >= 0, x, 0.2 * x)


def _iota2(shape, dim):
    return jax.lax.broadcasted_iota(jnp.int32, shape, dim)


def _body(sig_ref, w1r_ref, fc1b_ref, fc2W_ref, fc2b_ref,
          attnW_ref, attna_ref, Wih_ref, Whh_ref, bih_ref, bhh_ref,
          outW_ref, outb_ref, out_ref):
    f32 = jnp.float32
    bf16 = jnp.bfloat16
    hi = jax.lax.Precision.HIGHEST

    sig = sig_ref[...]
    w1r = w1r_ref[...]
    fc1b = fc1b_ref[...]
    fc2W = fc2W_ref[...]
    fc2b = fc2b_ref[...]
    attnW = attnW_ref[...]
    attna = attna_ref[...]
    Wih = Wih_ref[...]
    Whh = Whh_ref[...]
    bih = bih_ref[...]
    bhh = bhh_ref[...]
    outW = outW_ref[...]
    outb = outb_ref[...]

    def dotb(x, W):
        return jax.lax.dot_general(
            x.astype(bf16), W.astype(bf16),
            (((x.ndim - 1,), (1,)), ((), ())),
            preferred_element_type=f32)

    P1 = jnp.where(_iota2((V, 8), 0) // 16 == _iota2((V, 8), 1),
                   f32(1.0 / 16.0), f32(0.0))
    P2 = jnp.where(_iota2((V, 8), 0) % 8 == _iota2((V, 8), 1),
                   f32(1.0 / 16.0), f32(0.0))
    m1 = jnp.dot(sig, P1, precision=hi, preferred_element_type=f32)
    m2 = jnp.dot(sig, P2, precision=hi, preferred_element_type=f32)

    h1_0 = m1[:, :, None] * w1r[None] + fc1b[None]
    h1_1 = m2[:, :, None] * w1r[None] + fc1b[None]
    G0 = dotb(h1_0, fc2W) + fc2b[None]
    G1 = dotb(h1_1, fc2W) + fc2b[None]

    X0 = jnp.concatenate(
        [jnp.broadcast_to(G0[:, q:q + 1, :], (NR, 8, HID)) for q in range(8)],
        axis=1)
    X1 = jnp.concatenate([G1] * 8, axis=1)

    refm = 0.5 * (X0 + X1)
    refw = dotb(refm, attnW)
    a1 = attna[:, :HID]
    a2 = attna[:, HID:]
    zx0s = dotb(dotb(G0, attnW), a1)
    zx1s = dotb(dotb(G1, attnW), a1)
    zx0 = jnp.concatenate(
        [jnp.broadcast_to(zx0s[:, q:q + 1, :], (NR, 8, 1)) for q in range(8)],
        axis=1)
    zx1 = jnp.concatenate([zx1s] * 8, axis=1)
    zr = dotb(refw, a2)
    z0 = _leaky(zx0 + zr)
    z1 = _leaky(zx1 + zr)
    al0 = jax.nn.sigmoid(z0 - z1)
    Y = al0 * X0 + (1.0 - al0) * X1

    gb = (bih + bhh)[None]

    xWs = [dotb(Y[t * B:(t + 1) * B], Wih) + gb for t in range(M)]
    xWp = [dotb(Y[M * B + t * B: M * B + (t + 1) * B], Wih) + gb
           for t in range(N_P)]

    h = jnp.zeros((2 * B, 64, HID), f32)
    cc = jnp.zeros((2 * B, 64, HID), f32)

    def lstm_update(gates, cc):
        i = jax.nn.sigmoid(gates[..., 0 * HID:1 * HID])
        f = jax.nn.sigmoid(gates[..., 1 * HID:2 * HID])
        g = jnp.tanh(gates[..., 2 * HID:3 * HID])
        o = jax.nn.sigmoid(gates[..., 3 * HID:4 * HID])
        cc = f * cc + i * g
        return o * jnp.tanh(cc), cc

    for t in range(4):
        gates = xWs[t] + dotb(h[:B], Whh)
        h_new, cc_new = lstm_update(gates, cc[:B])
        h = jnp.concatenate([h_new, h[B:]], axis=0)
        cc = jnp.concatenate([cc_new, cc[B:]], axis=0)
    for t in range(4, 8):
        x_t = jnp.concatenate([xWs[t], xWp[t - 4]], axis=0)
        gates = x_t + dotb(h, Whh)
        h, cc = lstm_update(gates, cc)

    h_t = h[:B]
    h_p = h[B:]

    o1 = dotb(h_t, outW[:, :HID])
    o2 = dotb(h_p, outW[:, HID:])
    O = jnp.sum(o1 + o2 + outb[0, 0], axis=2)

    vcol = _iota2((64, V), 1)
    Xp = jnp.where(8 * (vcol // 16) + vcol % 8 == _iota2((64, V), 0),
                   f32(1.0), f32(0.0))
    out_ref[...] = jnp.dot(O, Xp, precision=hi, preferred_element_type=f32)


def kernel(tendency, periodicity, fc1_W, fc1_b, fc2_W, fc2_b, attn_W, attn_a,
           lstm_Wih, lstm_Whh, lstm_bih, lstm_bhh, out_W, out_b):
    f32 = jnp.float32
    sig = jnp.concatenate([
        jnp.transpose(tendency, (1, 0, 2)).reshape(M * B, V),
        jnp.transpose(periodicity, (1, 0, 2)).reshape(N_P * B, V)],
        axis=0).astype(f32)
    args = (
        sig,
        fc1_W.reshape(1, HID).astype(f32),
        fc1_b.reshape(1, HID).astype(f32),
        fc2_W.astype(f32),
        fc2_b.reshape(1, HID).astype(f32),
        attn_W.astype(f32),
        attn_a.reshape(1, 2 * HID).astype(f32),
        lstm_Wih.astype(f32),
        lstm_Whh.astype(f32),
        lstm_bih.reshape(1, 4 * HID).astype(f32),
        lstm_bhh.reshape(1, 4 * HID).astype(f32),
        out_W.reshape(1, 2 * HID).astype(f32),
        out_b.reshape(1, 1).astype(f32),
    )
    out = pl.pallas_call(
        _body,
        out_shape=jax.ShapeDtypeStruct((B, V), f32),
    )(*args)
    return out


# speedup vs baseline: 1.1397x; 1.0151x over previous
import jax
import jax.numpy as jnp
from jax.experimental import pallas as pl

V = 128
HID = 64
M = 8
N_P = 4
B = 16
NR = (M + N_P) * B


def _leaky(x):
    return jnp.where(x >= 0, x, 0.2 * x)


def _iota2(shape, dim):
    return jax.lax.broadcasted_iota(jnp.int32, shape, dim)


def _body(sig_ref, w1r_ref, fc1b_ref, fc2W_ref, fc2b_ref,
          attnW_ref, attna_ref, Wih_ref, Whh_ref, bih_ref, bhh_ref,
          outW_ref, outb_ref, out_ref):
    f32 = jnp.float32
    bf16 = jnp.bfloat16
    hi = jax.lax.Precision.HIGHEST

    sig = sig_ref[...]
    w1r = w1r_ref[...]
    fc1b = fc1b_ref[...]
    fc2W = fc2W_ref[...]
    fc2b = fc2b_ref[...]
    attnW = attnW_ref[...]
    attna = attna_ref[...]
    Wih = Wih_ref[...]
    Whh = Whh_ref[...]
    bih = bih_ref[...]
    bhh = bhh_ref[...]
    outW = outW_ref[...]
    outb = outb_ref[...]

    def dotb(x, W):
        return jax.lax.dot_general(
            x.astype(bf16), W.astype(bf16),
            (((x.ndim - 1,), (1,)), ((), ())),
            preferred_element_type=f32)

    P1 = jnp.where(_iota2((V, 8), 0) // 16 == _iota2((V, 8), 1),
                   f32(1.0 / 16.0), f32(0.0))
    P2 = jnp.where(_iota2((V, 8), 0) % 8 == _iota2((V, 8), 1),
                   f32(1.0 / 16.0), f32(0.0))
    m1 = jnp.dot(sig, P1, precision=hi, preferred_element_type=f32)
    m2 = jnp.dot(sig, P2, precision=hi, preferred_element_type=f32)

    h1_0 = m1[:, :, None] * w1r[None] + fc1b[None]
    h1_1 = m2[:, :, None] * w1r[None] + fc1b[None]
    G0 = dotb(h1_0, fc2W) + fc2b[None]
    G1 = dotb(h1_1, fc2W) + fc2b[None]

    X0 = jnp.concatenate(
        [jnp.broadcast_to(G0[:, q:q + 1, :], (NR, 8, HID)) for q in range(8)],
        axis=1)
    X1 = jnp.concatenate([G1] * 8, axis=1)

    refm = 0.5 * (X0 + X1)
    refw = dotb(refm, attnW)
    a1 = attna[:, :HID]
    a2 = attna[:, HID:]
    zx0s = dotb(dotb(G0, attnW), a1)
    zx1s = dotb(dotb(G1, attnW), a1)
    zx0 = jnp.concatenate(
        [jnp.broadcast_to(zx0s[:, q:q + 1, :], (NR, 8, 1)) for q in range(8)],
        axis=1)
    zx1 = jnp.concatenate([zx1s] * 8, axis=1)
    zr = dotb(refw, a2)
    z0 = _leaky(zx0 + zr)
    z1 = _leaky(zx1 + zr)
    al0 = jax.nn.sigmoid(z0 - z1)
    Y = al0 * X0 + (1.0 - al0) * X1

    gb = (bih + bhh)[None]

    xWs = [dotb(Y[t * B:(t + 1) * B], Wih) + gb for t in range(M)]
    xWp = [dotb(Y[M * B + t * B: M * B + (t + 1) * B], Wih) + gb
           for t in range(N_P)]

    h = jnp.zeros((2 * B, 64, HID), f32)
    cc = jnp.zeros((2 * B, 64, HID), f32)

    def lstm_update(gates, cc):
        i = jax.nn.sigmoid(gates[..., 0 * HID:1 * HID])
        f = jax.nn.sigmoid(gates[..., 1 * HID:2 * HID])
        g = jnp.tanh(gates[..., 2 * HID:3 * HID])
        o = jax.nn.sigmoid(gates[..., 3 * HID:4 * HID])
        cc = f * cc + i * g
        return o * jnp.tanh(cc), cc

    h0, c0 = lstm_update(xWs[0], cc[:B])
    h = jnp.concatenate([h0, h[B:]], axis=0)
    cc = jnp.concatenate([c0, cc[B:]], axis=0)
    for t in range(1, 4):
        gates = xWs[t] + dotb(h[:B], Whh)
        h_new, cc_new = lstm_update(gates, cc[:B])
        h = jnp.concatenate([h_new, h[B:]], axis=0)
        cc = jnp.concatenate([cc_new, cc[B:]], axis=0)
    gates4 = jnp.concatenate([xWs[4] + dotb(h[:B], Whh), xWp[0]], axis=0)
    h, cc = lstm_update(gates4, cc)
    for t in range(5, 8):
        x_t = jnp.concatenate([xWs[t], xWp[t - 4]], axis=0)
        gates = x_t + dotb(h, Whh)
        h, cc = lstm_update(gates, cc)

    h_t = h[:B]
    h_p = h[B:]

    o1 = dotb(h_t, outW[:, :HID])
    o2 = dotb(h_p, outW[:, HID:])
    O = jnp.sum(o1 + o2 + outb[0, 0], axis=2)

    vcol = _iota2((64, V), 1)
    Xp = jnp.where(8 * (vcol // 16) + vcol % 8 == _iota2((64, V), 0),
                   f32(1.0), f32(0.0))
    out_ref[...] = jnp.dot(O, Xp, precision=hi, preferred_element_type=f32)


def kernel(tendency, periodicity, fc1_W, fc1_b, fc2_W, fc2_b, attn_W, attn_a,
           lstm_Wih, lstm_Whh, lstm_bih, lstm_bhh, out_W, out_b):
    f32 = jnp.float32
    sig = jnp.concatenate([
        jnp.transpose(tendency, (1, 0, 2)).reshape(M * B, V),
        jnp.transpose(periodicity, (1, 0, 2)).reshape(N_P * B, V)],
        axis=0).astype(f32)
    args = (
        sig,
        fc1_W.reshape(1, HID).astype(f32),
        fc1_b.reshape(1, HID).astype(f32),
        fc2_W.astype(f32),
        fc2_b.reshape(1, HID).astype(f32),
        attn_W.astype(f32),
        attn_a.reshape(1, 2 * HID).astype(f32),
        lstm_Wih.astype(f32),
        lstm_Whh.astype(f32),
        lstm_bih.reshape(1, 4 * HID).astype(f32),
        lstm_bhh.reshape(1, 4 * HID).astype(f32),
        out_W.reshape(1, 2 * HID).astype(f32),
        out_b.reshape(1, 1).astype(f32),
    )
    out = pl.pallas_call(
        _body,
        out_shape=jax.ShapeDtypeStruct((B, V), f32),
    )(*args)
    return out
